# Initial kernel scaffold; baseline (speedup 1.0000x reference)
import jax
import jax.numpy as jnp
from jax.experimental import pallas as pl


def kernel(x_nchw, w1, b1, w2, b2, ws, bs):
    raise NotImplementedError("write your pallas kernel here")


# speedup vs baseline: 1.2337x; 1.2337x over previous
import jax
import jax.numpy as jnp
from jax.experimental import pallas as pl
from jax.experimental.pallas import tpu as pltpu


def _scse_plane_kernel(x_ref, w1t_ref, b1t_ref, w2t_ref, b2t_ref, wst_ref,
                       bs_ref, o_ref, *, inv_hw):
    xv = x_ref[0]

    pooled = jnp.sum(xv, axis=1, keepdims=True,
                     dtype=jnp.float32) * inv_hw
    h = jnp.maximum(
        jnp.dot(w1t_ref[...], pooled,
                preferred_element_type=jnp.float32) + b1t_ref[...],
        0.0,
    )
    cgate = jax.nn.sigmoid(
        jnp.dot(w2t_ref[...], h,
                preferred_element_type=jnp.float32) + b2t_ref[...]
    )

    slogit = jnp.dot(wst_ref[...], xv,
                     preferred_element_type=jnp.float32) + bs_ref[0, 0]
    sgate = jax.nn.sigmoid(slogit)

    o_ref[0] = xv * (cgate + sgate)


def kernel(x_nchw, w1, b1, w2, b2, ws, bs):
    N, C, H, W = x_nchw.shape
    HW = H * W
    Cr = w1.shape[1]

    x = x_nchw.reshape(N, C, HW)
    w1t = w1.T
    b1t = b1.reshape(Cr, 1)
    w2t = w2.T
    b2t = b2.reshape(C, 1)
    wst = ws.reshape(1, C)
    bs2 = bs.reshape(1, 1)

    import functools
    body = functools.partial(_scse_plane_kernel, inv_hw=1.0 / float(HW))

    out = pl.pallas_call(
        body,
        out_shape=jax.ShapeDtypeStruct((N, C, HW), x.dtype),
        grid=(N,),
        in_specs=[
            pl.BlockSpec((1, C, HW), lambda n: (n, 0, 0)),
            pl.BlockSpec((Cr, C), lambda n: (0, 0)),
            pl.BlockSpec((Cr, 1), lambda n: (0, 0)),
            pl.BlockSpec((C, Cr), lambda n: (0, 0)),
            pl.BlockSpec((C, 1), lambda n: (0, 0)),
            pl.BlockSpec((1, C), lambda n: (0, 0)),
            pl.BlockSpec((1, 1), lambda n: (0, 0)),
        ],
        out_specs=pl.BlockSpec((1, C, HW), lambda n: (n, 0, 0)),
        compiler_params=pltpu.CompilerParams(
            dimension_semantics=("parallel",),
            vmem_limit_bytes=48 * 1024 * 1024,
        ),
    )(x, w1t, b1t, w2t, b2t, wst, bs2)
    return out.reshape(N, C, H, W)
